# baseline (speedup 1.0000x reference)
import functools

import jax
import jax.numpy as jnp
from jax import lax
from jax.experimental import pallas as pl
from jax.experimental.pallas import tpu as pltpu
from jax.experimental.pallas import tpu_sc as plsc

_B, _S, _D = 4, 2048, 1024
_P, _L, _K = 64, 8, 2

_TROWS = 1024
_SROWS = _S - _TROWS
_RBLK = 256
_NJ = _TROWS // _RBLK

_GRP = 8
_CW = _D // _GRP
_CH = 256
_NCH = _SROWS // _CH


def _sc_body(x_hbm, out_hbm, buf0, buf1, outv, sem0, sem1):
    wid = lax.axis_index("s") * 2 + lax.axis_index("c")
    seg = wid // _GRP
    w = wid % _GRP
    row0 = seg * _S + _TROWS
    col0 = w * _CW

    bufs = (buf0, buf1)
    sems = (sem0, sem1)
    pending = pltpu.async_copy(
        x_hbm.at[pl.ds(row0, _CH), pl.ds(col0, _CW)], buf0, sem0)
    acc = tuple(jnp.zeros((16,), jnp.float32) for _ in range(_CW // 16))
    for ci in range(_NCH):
        nxt = None
        if ci + 1 < _NCH:
            nxt = pltpu.async_copy(
                x_hbm.at[pl.ds(row0 + (ci + 1) * _CH, _CH),
                         pl.ds(col0, _CW)],
                bufs[(ci + 1) % 2], sems[(ci + 1) % 2])
        pending.wait()
        buf = bufs[ci % 2]

        def body(r, a):
            return tuple(a[c] + buf[r, pl.ds(c * 16, 16)]
                         for c in range(_CW // 16))

        acc = lax.fori_loop(0, _CH, body, acc)
        pending = nxt
    for c in range(_CW // 16):
        outv[0, pl.ds(c * 16, 16)] = acc[c]
    pltpu.sync_copy(outv, out_hbm.at[pl.ds(seg, 1), pl.ds(col0, _CW)])


_sc_partial = functools.partial(
    pl.kernel,
    out_type=jax.ShapeDtypeStruct((_B, _D), jnp.float32),
    mesh=plsc.VectorSubcoreMesh(core_axis_name="c", subcore_axis_name="s"),
    scratch_types=[
        pltpu.VMEM((_CH, _CW), jnp.float32),
        pltpu.VMEM((_CH, _CW), jnp.float32),
        pltpu.VMEM((1, _CW), jnp.float32),
        pltpu.SemaphoreType.DMA,
        pltpu.SemaphoreType.DMA,
    ],
)(_sc_body)


def _tc1_body(x_ref, out_ref, acc_ref):
    b = pl.program_id(0)
    j = pl.program_id(1)

    @pl.when((b == 0) & (j == 0))
    def _init():
        acc_ref[...] = jnp.zeros_like(acc_ref)

    partial = jnp.sum(x_ref[...], axis=0, keepdims=True)
    rowid = lax.broadcasted_iota(jnp.int32, (8, _D), 0)
    acc_ref[...] += jnp.where(rowid == b, partial, 0.0)

    @pl.when((b == _B - 1) & (j == _NJ - 1))
    def _out():
        out_ref[...] = acc_ref[...]


def _tc2_body(ptc_ref, psc_ref, keys_ref, prompts_ref,
              sim_ref, sep_ref, vals_ref, bp_ref):
    xm = (ptc_ref[0:_B, :] + psc_ref[...]) * (1.0 / _S)
    xn = xm * lax.rsqrt(
        jnp.maximum(jnp.sum(xm * xm, axis=1, keepdims=True), 1e-12))
    k = keys_ref[...]
    kn = k * lax.rsqrt(
        jnp.maximum(jnp.sum(k * k, axis=1, keepdims=True), 1e-12))
    sim = lax.dot_general(xn, kn, (((1,), (1,)), ((), ())),
                          preferred_element_type=jnp.float32)
    sim_ref[...] = sim
    gram = lax.dot_general(kn, kn, (((1,), (1,)), ((), ())),
                           preferred_element_type=jnp.float32)
    r = lax.broadcasted_iota(jnp.int32, (_P, _P), 0)
    c = lax.broadcasted_iota(jnp.int32, (_P, _P), 1)
    diff = gram - (r == c).astype(jnp.float32)
    sep_ref[...] = (jnp.sum(diff * diff) * (1.0 / (_P * _P))).reshape(1, 1)

    col = lax.broadcasted_iota(jnp.int32, (_B, _P), 1)
    v1 = jnp.max(sim, axis=1, keepdims=True)
    i1 = jnp.min(jnp.where(sim == v1, col, _P), axis=1, keepdims=True)
    masked = jnp.where(col == i1, -jnp.inf, sim)
    v2 = jnp.max(masked, axis=1, keepdims=True)
    i2 = jnp.min(jnp.where(masked == v2, col, _P), axis=1, keepdims=True)
    vals_ref[...] = jnp.concatenate([v1, v2], axis=1)

    for bb in range(_B):
        for kk in range(_K):
            idx = (i1 if kk == 0 else i2)[bb, 0]
            bp_ref[bb, kk * _L:(kk + 1) * _L, :] = prompts_ref[idx]


def kernel(x_embed, prompt_keys, prompts, layer_idx):
    x_flat = x_embed.reshape(_B * _S, _D)

    psc = _sc_partial(x_flat)

    ptc = pl.pallas_call(
        _tc1_body,
        grid=(_B, _NJ),
        in_specs=[
            pl.BlockSpec((_RBLK, _D),
                         lambda b, j: (b * (_S // _RBLK) + j, 0)),
        ],
        out_specs=pl.BlockSpec((8, _D), lambda b, j: (0, 0)),
        out_shape=jax.ShapeDtypeStruct((8, _D), jnp.float32),
        scratch_shapes=[pltpu.VMEM((8, _D), jnp.float32)],
        compiler_params=pltpu.CompilerParams(
            dimension_semantics=("arbitrary", "arbitrary")),
    )(x_flat)

    sim, sep, vals, bp = pl.pallas_call(
        _tc2_body,
        out_shape=[
            jax.ShapeDtypeStruct((_B, _P), jnp.float32),
            jax.ShapeDtypeStruct((1, 1), jnp.float32),
            jax.ShapeDtypeStruct((_B, _K), jnp.float32),
            jax.ShapeDtypeStruct((_B, _K * _L, _D), jnp.float32),
        ],
    )(ptc, psc, prompt_keys, prompts)

    orth = jnp.zeros((), jnp.float32)
    return (sim, orth, sep.reshape(()), vals, bp)


# speedup vs baseline: 2.0901x; 2.0901x over previous
import functools

import jax
import jax.numpy as jnp
from jax import lax
from jax.experimental import pallas as pl
from jax.experimental.pallas import tpu as pltpu

_B, _S, _D = 4, 2048, 1024
_P, _L, _K = 64, 8, 2
_RBLK = 1024
_NBLK = _B * _S // _RBLK
_BLK_PER_B = _S // _RBLK


def _body(x_ref, keys_ref, prompts_hbm, sim_ref, sep_ref, vals_ref, bp_ref,
          acc_ref, kn_ref, sem):
    i = pl.program_id(0)

    @pl.when(i == 0)
    def _init():
        acc_ref[...] = jnp.zeros_like(acc_ref)
        k = keys_ref[...]
        kn = k * lax.rsqrt(
            jnp.maximum(jnp.sum(k * k, axis=1, keepdims=True), 1e-12))
        kn_ref[...] = kn
        gram = lax.dot_general(kn, kn, (((1,), (1,)), ((), ())),
                               preferred_element_type=jnp.float32)
        r = lax.broadcasted_iota(jnp.int32, (_P, _P), 0)
        c = lax.broadcasted_iota(jnp.int32, (_P, _P), 1)
        diff = gram - (r == c).astype(jnp.float32)
        sep_ref[...] = (jnp.sum(diff * diff) * (1.0 / (_P * _P))).reshape(1, 1)

    b = i // _BLK_PER_B
    partial = jnp.sum(x_ref[...], axis=0, keepdims=True)
    rowid = lax.broadcasted_iota(jnp.int32, (8, _D), 0)
    acc_ref[...] += jnp.where(rowid == b, partial, 0.0)

    @pl.when(i == _NBLK - 1)
    def _tail():
        xm = acc_ref[0:_B, :] * (1.0 / _S)
        xn = xm * lax.rsqrt(
            jnp.maximum(jnp.sum(xm * xm, axis=1, keepdims=True), 1e-12))
        sim = lax.dot_general(xn, kn_ref[...], (((1,), (1,)), ((), ())),
                              preferred_element_type=jnp.float32)
        sim_ref[...] = sim

        col = lax.broadcasted_iota(jnp.int32, (_B, _P), 1)
        v1 = jnp.max(sim, axis=1, keepdims=True)
        i1 = jnp.min(jnp.where(sim == v1, col, _P), axis=1, keepdims=True)
        masked = jnp.where(col == i1, -jnp.inf, sim)
        v2 = jnp.max(masked, axis=1, keepdims=True)
        i2 = jnp.min(jnp.where(masked == v2, col, _P), axis=1, keepdims=True)
        vals_ref[...] = jnp.concatenate([v1, v2], axis=1)

        copies = []
        for bb in range(_B):
            for kk in range(_K):
                idx = (i1 if kk == 0 else i2)[bb, 0]
                copies.append(pltpu.make_async_copy(
                    prompts_hbm.at[idx],
                    bp_ref.at[bb, pl.ds(kk * _L, _L), :],
                    sem))
        for cp in copies:
            cp.start()
        for cp in copies:
            cp.wait()


def kernel(x_embed, prompt_keys, prompts, layer_idx):
    x_flat = x_embed.reshape(_B * _S, _D)
    sim, sep, vals, bp = pl.pallas_call(
        _body,
        grid=(_NBLK,),
        in_specs=[
            pl.BlockSpec((_RBLK, _D), lambda i: (i, 0)),
            pl.BlockSpec((_P, _D), lambda i: (0, 0)),
            pl.BlockSpec(memory_space=pltpu.MemorySpace.HBM),
        ],
        out_specs=[
            pl.BlockSpec((_B, _P), lambda i: (0, 0)),
            pl.BlockSpec((1, 1), lambda i: (0, 0)),
            pl.BlockSpec((_B, _K), lambda i: (0, 0)),
            pl.BlockSpec((_B, _K * _L, _D), lambda i: (0, 0, 0)),
        ],
        out_shape=[
            jax.ShapeDtypeStruct((_B, _P), jnp.float32),
            jax.ShapeDtypeStruct((1, 1), jnp.float32),
            jax.ShapeDtypeStruct((_B, _K), jnp.float32),
            jax.ShapeDtypeStruct((_B, _K * _L, _D), jnp.float32),
        ],
        scratch_shapes=[
            pltpu.VMEM((8, _D), jnp.float32),
            pltpu.VMEM((_P, _D), jnp.float32),
            pltpu.SemaphoreType.DMA,
        ],
        compiler_params=pltpu.CompilerParams(
            dimension_semantics=("arbitrary",)),
    )(x_flat, prompt_keys, prompts)
    orth = jnp.zeros((), jnp.float32)
    return (sim, orth, sep.reshape(()), vals, bp)


# speedup vs baseline: 2.1392x; 1.0235x over previous
import functools

import jax
import jax.numpy as jnp
from jax.experimental import pallas as pl
from jax.experimental.pallas import tpu as pltpu

_B, _S, _D = 4, 2048, 1024
_P, _L, _K = 64, 8, 2
_RBLK = 1024
_NBLK = _B * _S // _RBLK
_BLK_PER_B = _S // _RBLK


def _body(x_ref, keys_ref, prompts_ref, sim_ref, sep_ref, vals_ref, bp_ref,
          acc_ref, kn_ref):
    i = pl.program_id(0)

    @pl.when(i == 0)
    def _init():
        acc_ref[...] = jnp.zeros_like(acc_ref)
        k = keys_ref[...]
        kn = k * jax.lax.rsqrt(
            jnp.maximum(jnp.sum(k * k, axis=1, keepdims=True), 1e-12))
        kn_ref[...] = kn
        gram = jax.lax.dot_general(kn, kn, (((1,), (1,)), ((), ())),
                                   preferred_element_type=jnp.float32)
        r = jax.lax.broadcasted_iota(jnp.int32, (_P, _P), 0)
        c = jax.lax.broadcasted_iota(jnp.int32, (_P, _P), 1)
        diff = gram - (r == c).astype(jnp.float32)
        sep_ref[...] = (jnp.sum(diff * diff) * (1.0 / (_P * _P))).reshape(1, 1)

    b = i // _BLK_PER_B
    partial = jnp.sum(x_ref[...], axis=0, keepdims=True)
    rowid = jax.lax.broadcasted_iota(jnp.int32, (8, _D), 0)
    acc_ref[...] += jnp.where(rowid == b, partial, 0.0)

    @pl.when(i == _NBLK - 1)
    def _tail():
        xm = acc_ref[0:_B, :] * (1.0 / _S)
        xn = xm * jax.lax.rsqrt(
            jnp.maximum(jnp.sum(xm * xm, axis=1, keepdims=True), 1e-12))
        sim = jax.lax.dot_general(xn, kn_ref[...], (((1,), (1,)), ((), ())),
                                  preferred_element_type=jnp.float32)
        sim_ref[...] = sim

        col = jax.lax.broadcasted_iota(jnp.int32, (_B, _P), 1)
        v1 = jnp.max(sim, axis=1, keepdims=True)
        i1 = jnp.min(jnp.where(sim == v1, col, _P), axis=1, keepdims=True)
        masked = jnp.where(col == i1, -jnp.inf, sim)
        v2 = jnp.max(masked, axis=1, keepdims=True)
        i2 = jnp.min(jnp.where(masked == v2, col, _P), axis=1, keepdims=True)
        vals_ref[...] = jnp.concatenate([v1, v2], axis=1)

        for bb in range(_B):
            for kk in range(_K):
                idx = (i1 if kk == 0 else i2)[bb, 0]
                bp_ref[bb, kk * _L:(kk + 1) * _L, :] = prompts_ref[idx]


def kernel(x_embed, prompt_keys, prompts, layer_idx):
    x_flat = x_embed.reshape(_B * _S, _D)
    sim, sep, vals, bp = pl.pallas_call(
        _body,
        grid=(_NBLK,),
        in_specs=[
            pl.BlockSpec((_RBLK, _D), lambda i: (i, 0)),
            pl.BlockSpec((_P, _D), lambda i: (0, 0)),
            pl.BlockSpec((_P, _L, _D), lambda i: (0, 0, 0)),
        ],
        out_specs=[
            pl.BlockSpec((_B, _P), lambda i: (0, 0)),
            pl.BlockSpec((1, 1), lambda i: (0, 0)),
            pl.BlockSpec((_B, _K), lambda i: (0, 0)),
            pl.BlockSpec((_B, _K * _L, _D), lambda i: (0, 0, 0)),
        ],
        out_shape=[
            jax.ShapeDtypeStruct((_B, _P), jnp.float32),
            jax.ShapeDtypeStruct((1, 1), jnp.float32),
            jax.ShapeDtypeStruct((_B, _K), jnp.float32),
            jax.ShapeDtypeStruct((_B, _K * _L, _D), jnp.float32),
        ],
        scratch_shapes=[pltpu.VMEM((8, _D), jnp.float32),
                        pltpu.VMEM((_P, _D), jnp.float32)],
        compiler_params=pltpu.CompilerParams(
            dimension_semantics=("arbitrary",)),
    )(x_flat, prompt_keys, prompts)
    orth = jnp.zeros((), jnp.float32)
    return (sim, orth, sep.reshape(()), vals, bp)


# speedup vs baseline: 2.2761x; 1.0640x over previous
import functools

import jax
import jax.numpy as jnp
from jax.experimental import pallas as pl
from jax.experimental.pallas import tpu as pltpu

_B, _S, _D = 4, 2048, 1024
_P, _L, _K = 64, 8, 2
_RBLK = 2048
_NBLK = _B * _S // _RBLK
_BLK_PER_B = _S // _RBLK


def _body(x_ref, keys_ref, prompts_ref, sim_ref, sep_ref, vals_ref, bp_ref,
          acc_ref):
    i = pl.program_id(0)

    @pl.when(i == 0)
    def _init():
        acc_ref[...] = jnp.zeros_like(acc_ref)

    b = i // _BLK_PER_B
    partial = jnp.sum(x_ref[...], axis=0, keepdims=True)
    rowid = jax.lax.broadcasted_iota(jnp.int32, (8, _D), 0)
    acc_ref[...] += jnp.where(rowid == b, partial, 0.0)

    @pl.when(i == _NBLK - 1)
    def _tail():
        xm = acc_ref[0:_B, :] * (1.0 / _S)
        xn = xm * jax.lax.rsqrt(
            jnp.maximum(jnp.sum(xm * xm, axis=1, keepdims=True), 1e-12))
        k = keys_ref[...]
        kn = k * jax.lax.rsqrt(
            jnp.maximum(jnp.sum(k * k, axis=1, keepdims=True), 1e-12))
        sim = jax.lax.dot_general(xn, kn, (((1,), (1,)), ((), ())),
                                  preferred_element_type=jnp.float32)
        sim_ref[...] = sim
        gram = jax.lax.dot_general(kn, kn, (((1,), (1,)), ((), ())),
                                   preferred_element_type=jnp.float32)
        r = jax.lax.broadcasted_iota(jnp.int32, (_P, _P), 0)
        c = jax.lax.broadcasted_iota(jnp.int32, (_P, _P), 1)
        diff = gram - (r == c).astype(jnp.float32)
        sep_ref[...] = (jnp.sum(diff * diff) * (1.0 / (_P * _P))).reshape(1, 1)

        col = jax.lax.broadcasted_iota(jnp.int32, (_B, _P), 1)
        v1 = jnp.max(sim, axis=1, keepdims=True)
        i1 = jnp.min(jnp.where(sim == v1, col, _P), axis=1, keepdims=True)
        masked = jnp.where(col == i1, -jnp.inf, sim)
        v2 = jnp.max(masked, axis=1, keepdims=True)
        i2 = jnp.min(jnp.where(masked == v2, col, _P), axis=1, keepdims=True)
        vals_ref[...] = jnp.concatenate([v1, v2], axis=1)

        for bb in range(_B):
            for kk in range(_K):
                idx = (i1 if kk == 0 else i2)[bb, 0]
                bp_ref[bb, kk * _L:(kk + 1) * _L, :] = prompts_ref[idx]


def kernel(x_embed, prompt_keys, prompts, layer_idx):
    x_flat = x_embed.reshape(_B * _S, _D)
    sim, sep, vals, bp = pl.pallas_call(
        _body,
        grid=(_NBLK,),
        in_specs=[
            pl.BlockSpec((_RBLK, _D), lambda i: (i, 0)),
            pl.BlockSpec((_P, _D), lambda i: (0, 0)),
            pl.BlockSpec((_P, _L, _D), lambda i: (0, 0, 0)),
        ],
        out_specs=[
            pl.BlockSpec((_B, _P), lambda i: (0, 0)),
            pl.BlockSpec((1, 1), lambda i: (0, 0)),
            pl.BlockSpec((_B, _K), lambda i: (0, 0)),
            pl.BlockSpec((_B, _K * _L, _D), lambda i: (0, 0, 0)),
        ],
        out_shape=[
            jax.ShapeDtypeStruct((_B, _P), jnp.float32),
            jax.ShapeDtypeStruct((1, 1), jnp.float32),
            jax.ShapeDtypeStruct((_B, _K), jnp.float32),
            jax.ShapeDtypeStruct((_B, _K * _L, _D), jnp.float32),
        ],
        scratch_shapes=[pltpu.VMEM((8, _D), jnp.float32)],
        compiler_params=pltpu.CompilerParams(
            dimension_semantics=("arbitrary",)),
    )(x_flat, prompt_keys, prompts)
    orth = jnp.zeros((), jnp.float32)
    return (sim, orth, sep.reshape(()), vals, bp)
